# baseline (speedup 1.0000x reference)
import jax
import jax.numpy as jnp
from jax import lax
from jax.experimental import pallas as pl
from jax.experimental.pallas import tpu as pltpu
from jax.experimental.pallas import tpu_sc as plsc

B, C, H, W = 2, 96, 384, 384
QK_DIM = 96
NUM_HEADS = 3
K_SP = 576
TOPK = 64
HEAD_DIM = QK_DIM // NUM_HEADS
SC_SCALE = HEAD_DIM ** (-0.5)
HW = H * W
N = K_SP * TOPK
BN = B * N

NC, NS = 2, 16
NW = NC * NS

_HB1 = 8
_T1 = _HB1 * W


def _k1_body(x_ref, lnw_ref, lnb_ref, qw_ref, kw_ref, vw_ref, qk_ref, v_ref):
    x2 = jnp.reshape(x_ref[0], (C, _T1))
    xt = jnp.transpose(x2, (1, 0))
    mu = jnp.mean(xt, axis=1, keepdims=True)
    var = jnp.mean((xt - mu) ** 2, axis=1, keepdims=True)
    xn = (xt - mu) * lax.rsqrt(var + 1e-6)
    xn = xn * lnw_ref[:] + lnb_ref[:]
    dn = (((1,), (1,)), ((), ()))
    q = lax.dot_general(xn, qw_ref[:], dn, preferred_element_type=jnp.float32)
    k = lax.dot_general(xn, kw_ref[:], dn, preferred_element_type=jnp.float32)
    v = lax.dot_general(xn, vw_ref[:], dn, preferred_element_type=jnp.float32)
    z32 = jnp.zeros((_T1, 32), jnp.float32)
    qk_ref[...] = jnp.concatenate([q, z32, k, z32], axis=1)
    v_ref[...] = jnp.concatenate([v, z32], axis=1)


def _ln_qkv(x, ln_w2, ln_b2, q_w, k_w, v_w):
    grid = (B, H // _HB1)
    return pl.pallas_call(
        _k1_body,
        grid=grid,
        in_specs=[
            pl.BlockSpec((1, C, _HB1, W), lambda b, t: (b, 0, t, 0)),
            pl.BlockSpec((1, C), lambda b, t: (0, 0)),
            pl.BlockSpec((1, C), lambda b, t: (0, 0)),
            pl.BlockSpec((C, C), lambda b, t: (0, 0)),
            pl.BlockSpec((C, C), lambda b, t: (0, 0)),
            pl.BlockSpec((C, C), lambda b, t: (0, 0)),
        ],
        out_specs=[
            pl.BlockSpec((_T1, 256), lambda b, t: (b * (H // _HB1) + t, 0)),
            pl.BlockSpec((_T1, 128), lambda b, t: (b * (H // _HB1) + t, 0)),
        ],
        out_shape=[
            jax.ShapeDtypeStruct((B * HW, 256), jnp.float32),
            jax.ShapeDtypeStruct((B * HW, 128), jnp.float32),
        ],
        compiler_params=pltpu.CompilerParams(
            dimension_semantics=("parallel", "parallel")),
    )(x, ln_w2, ln_b2, q_w, k_w, v_w)


_G_ROWS = BN // NW
_G_CH = 96


def _k2_body(qk_hbm, v_hbm, gidx_hbm, qkg_hbm, vg_hbm, idx_v, qbuf, vbuf, sem):
    wid = lax.axis_index("s") * NC + lax.axis_index("c")
    base = wid * _G_ROWS
    pltpu.sync_copy(gidx_hbm.at[pl.ds(base, _G_ROWS)], idx_v)

    def chunk(i, carry):
        off = i * _G_CH
        pltpu.async_copy(qk_hbm.at[idx_v.at[pl.ds(off, _G_CH)]], qbuf, sem).wait()
        pltpu.sync_copy(qbuf, qkg_hbm.at[pl.ds(base + off, _G_CH)])
        pltpu.async_copy(v_hbm.at[idx_v.at[pl.ds(off, _G_CH)]], vbuf, sem).wait()
        pltpu.sync_copy(vbuf, vg_hbm.at[pl.ds(base + off, _G_CH)])
        return carry

    lax.fori_loop(0, _G_ROWS // _G_CH, chunk, 0)


def _sc_gather(qk_tab, v_tab, gidx):
    mesh = plsc.VectorSubcoreMesh(core_axis_name="c", subcore_axis_name="s")
    f = pl.kernel(
        _k2_body,
        out_type=[
            jax.ShapeDtypeStruct((BN, 256), jnp.float32),
            jax.ShapeDtypeStruct((BN, 128), jnp.float32),
        ],
        mesh=mesh,
        scratch_types=[
            pltpu.VMEM((_G_ROWS,), jnp.int32),
            pltpu.VMEM((_G_CH, 256), jnp.float32),
            pltpu.VMEM((_G_CH, 128), jnp.float32),
            pltpu.SemaphoreType.DMA,
        ],
        compiler_params=pltpu.CompilerParams(use_tc_tiling_on_sc=True),
    )
    return f(qk_tab, v_tab, gidx)


_GSP = 2
_RWS = _GSP * TOPK


def _k3_body(qk_ref, v_ref, sims_ref, out_ref, pen_sc):
    X = qk_ref[...]
    XV = v_ref[...]
    sT = jnp.transpose(sims_ref[0], (1, 0))
    ones1 = jnp.ones((_RWS, 1), jnp.float32)

    @pl.when(pl.program_id(0) == 0)
    def _():
        rb = lax.shift_right_logical(
            lax.broadcasted_iota(jnp.int32, (_RWS, _RWS), 0), 6)
        cb = lax.shift_right_logical(
            lax.broadcasted_iota(jnp.int32, (_RWS, _RWS), 1), 6)
        pen_sc[...] = jnp.where(rb == cb, 0.0, -1e30)

    pen = pen_sc[...]
    outs = []
    dn = (((1,), (1,)), ((), ()))
    for h in range(NUM_HEADS):
        q = X[:, h * HEAD_DIM:(h + 1) * HEAD_DIM]
        k = X[:, 128 + h * HEAD_DIM:128 + (h + 1) * HEAD_DIM]
        v = XV[:, h * HEAD_DIM:(h + 1) * HEAD_DIM]
        qn = jnp.sum(q * q, axis=1, keepdims=True)
        kn = jnp.sum(k * k, axis=1, keepdims=True)
        qa = jnp.concatenate([-2.0 * q, ones1], axis=1)
        ka = jnp.concatenate([k, kn], axis=1)
        d2k = lax.dot_general(qa, ka, dn,
                              preferred_element_type=jnp.float32)
        d2 = d2k + qn
        dist = jnp.sqrt(jnp.maximum(d2, 1e-12))
        e = jnp.exp(pen - SC_SCALE * dist)
        vwa = jnp.concatenate([v * sT, ones1], axis=1)
        dn2 = (((1,), (0,)), ((), ()))
        os = lax.dot_general(e, vwa, dn2,
                             preferred_element_type=jnp.float32)
        outs.append(os[:, 0:HEAD_DIM] * (sT / os[:, HEAD_DIM:HEAD_DIM + 1]))
    z31 = jnp.zeros((_RWS, 31), jnp.float32)
    out_ref[...] = jnp.concatenate(outs + [ones1, z31], axis=1)


def _attention(qk_g, v_g, sims2):
    grid = (BN // _RWS,)
    return pl.pallas_call(
        _k3_body,
        grid=grid,
        in_specs=[
            pl.BlockSpec((_RWS, 256), lambda i: (i, 0)),
            pl.BlockSpec((_RWS, 128), lambda i: (i, 0)),
            pl.BlockSpec((1, 1, _RWS), lambda i: (i, 0, 0)),
        ],
        out_specs=pl.BlockSpec((_RWS, 128), lambda i: (i, 0)),
        out_shape=jax.ShapeDtypeStruct((BN, 128), jnp.float32),
        scratch_shapes=[pltpu.VMEM((_RWS, _RWS), jnp.float32)],
        compiler_params=pltpu.CompilerParams(
            dimension_semantics=("arbitrary",)),
    )(qk_g, v_g, sims2)


_R = 12288
_NRANGE = HW // _R
_S_TOK = N // NS
_S_CH = 128
_ZROWS = 32


def _k4_body(tok_hbm, gidx_hbm, acc_hbm,
             idxa_v, cidx, cdst, cidx_c, cdst_c, bufa, zbuf,
             sema, semz, acc_sp):
    c = lax.axis_index("c")
    s = lax.axis_index("s")
    base_tok = c * N + s * _S_TOK

    with jax.named_scope("k4_init"):
        def zfill(i, carry):
            def zf2(j, carry2):
                zbuf[i, pl.ds(j * 16, 16)] = jnp.zeros((16,), jnp.float32)
                return carry2
            lax.fori_loop(0, 128 // 16, zf2, 0)
            return carry
        lax.fori_loop(0, _ZROWS, zfill, 0)

    pltpu.sync_copy(gidx_hbm.at[pl.ds(base_tok, _S_TOK)], idxa_v)
    iota16 = lax.iota(jnp.int32, 16)

    def one_range(rr, carry):
        r0g = c * HW + rr * _R
        with jax.named_scope("k4_zero"):
            def za(i, carry2):
                pltpu.async_copy(
                    zbuf,
                    acc_sp.at[pl.ds(s * (_R // NS) + i * _ZROWS, _ZROWS)],
                    semz)
                return carry2
            lax.fori_loop(0, _R // NS // _ZROWS, za, 0)

        with jax.named_scope("k4_compact"):
            def cp(i, off):
                vv = idxa_v[pl.ds(i * 16, 16)] - r0g
                msk = (vv >= 0) & (vv < _R)
                rows = base_tok + i * 16 + iota16
                plsc.store_compressed(cidx.at[pl.ds(off, 16)], rows, mask=msk)
                plsc.store_compressed(cdst.at[pl.ds(off, 16)], vv, mask=msk)
                pc = plsc.all_reduce_population_count(msk)
                return off + pc[0]
            nc = lax.fori_loop(0, _S_TOK // 16, cp, 0)
            def pad(k, carry2):
                cidx[pl.ds(nc + k * 16, 16)] = jnp.full((16,), base_tok,
                                                        jnp.int32)
                cdst[pl.ds(nc + k * 16, 16)] = jnp.full((16,), _R, jnp.int32)
                return carry2
            lax.fori_loop(0, _S_CH // 16, pad, 0)

        with jax.named_scope("k4_zdrain"):
            def zd(i, carry2):
                pltpu.make_async_copy(
                    zbuf, acc_sp.at[pl.ds(s * (_R // NS), _ZROWS)],
                    semz).wait()
                return carry2
            lax.fori_loop(0, _R // NS // _ZROWS, zd, 0)
        plsc.subcore_barrier()

        with jax.named_scope("k4_chunks"):
            nch = lax.shift_right_logical(nc + (_S_CH - 1), 7)

            def chunk(j, carry2):
                def mv(k, carry3):
                    cidx_c[pl.ds(k * 16, 16)] = (
                        cidx[pl.ds(j * _S_CH + k * 16, 16)])
                    cdst_c[pl.ds(k * 16, 16)] = (
                        cdst[pl.ds(j * _S_CH + k * 16, 16)])
                    return carry3
                lax.fori_loop(0, _S_CH // 16, mv, 0)
                pltpu.async_copy(tok_hbm.at[cidx_c], bufa, sema).wait()
                pltpu.sync_copy(bufa, acc_sp.at[cdst_c], add=True)
                return carry2
            lax.fori_loop(0, nch, chunk, 0)
        plsc.subcore_barrier()

        with jax.named_scope("k4_out"):
            pltpu.sync_copy(acc_sp.at[pl.ds(s * (_R // NS), _R // NS)],
                            acc_hbm.at[pl.ds(r0g + s * (_R // NS), _R // NS)])
        plsc.subcore_barrier()
        return carry

    lax.fori_loop(0, _NRANGE, one_range, 0)


def _sc_scatter(out_tok, gidx):
    mesh = plsc.VectorSubcoreMesh(core_axis_name="c", subcore_axis_name="s")
    f = pl.kernel(
        _k4_body,
        out_type=jax.ShapeDtypeStruct((B * HW, 128), jnp.float32),
        mesh=mesh,
        scratch_types=[
            pltpu.VMEM((_S_TOK,), jnp.int32),
            pltpu.VMEM((_S_TOK + _S_CH,), jnp.int32),
            pltpu.VMEM((_S_TOK + _S_CH,), jnp.int32),
            pltpu.VMEM((_S_CH,), jnp.int32),
            pltpu.VMEM((_S_CH,), jnp.int32),
            pltpu.VMEM((_S_CH, 128), jnp.float32),
            pltpu.VMEM((_ZROWS, 128), jnp.float32),
            pltpu.SemaphoreType.DMA,
            pltpu.SemaphoreType.DMA,
            pltpu.VMEM_SHARED((_R + 16, 128), jnp.float32),
        ],
        compiler_params=pltpu.CompilerParams(use_tc_tiling_on_sc=True,
                                             needs_layout_passes=False),
    )
    return f(out_tok, gidx)


_T5 = 12288
_HB5 = _T5 // W


def _k5_body(acc_ref, v_ref, out_ref):
    a = acc_ref[...]
    ct = a[:, C:C + 1]
    v = v_ref[:, 0:C]
    mean = a[:, 0:C] / jnp.maximum(ct, 1.0)
    res = jnp.where(ct > 1e-5, mean, v)
    rT = jnp.transpose(res, (1, 0))
    for hb in range(_HB5):
        out_ref[0, :, hb, :] = rT[:, hb * W:(hb + 1) * W]


def _merge(acc, v_tok):
    grid = (B * HW // _T5,)
    nh = H // _HB5
    return pl.pallas_call(
        _k5_body,
        grid=grid,
        in_specs=[
            pl.BlockSpec((_T5, 128), lambda t: (t, 0)),
            pl.BlockSpec((_T5, 128), lambda t: (t, 0)),
        ],
        out_specs=pl.BlockSpec((1, C, _HB5, W), lambda t: (t // nh, 0, t % nh, 0)),
        out_shape=jax.ShapeDtypeStruct((B, C, H, W), jnp.float32),
        compiler_params=pltpu.CompilerParams(
            dimension_semantics=("arbitrary",)),
    )(acc, v_tok)


@jax.jit
def _run(x, sims, ln_w, ln_b, q_w, k_w, v_w, indices):
    qk_tok, v_tok = _ln_qkv(x, ln_w.reshape(1, C), ln_b.reshape(1, C),
                            q_w, k_w, v_w)
    gidx = (indices.reshape(B, N)
            + (jnp.arange(B, dtype=jnp.int32) * HW)[:, None]).reshape(BN)
    qk_g, v_g = _sc_gather(qk_tok, v_tok, gidx)
    out_tok = _attention(qk_g, v_g, sims.reshape(BN // _RWS, 1, _RWS))
    acc = _sc_scatter(out_tok, gidx)
    return _merge(acc, v_tok)


def kernel(x, sims, mask, ln_w, ln_b, q_w, k_w, v_w, indices, labels,
           num_spixels):
    del mask, labels, num_spixels
    return _run(x, sims, ln_w, ln_b, q_w, k_w, v_w, indices)


# speedup vs baseline: 1.1328x; 1.1328x over previous
import jax
import jax.numpy as jnp
from jax import lax
from jax.experimental import pallas as pl
from jax.experimental.pallas import tpu as pltpu
from jax.experimental.pallas import tpu_sc as plsc

B, C, H, W = 2, 96, 384, 384
QK_DIM = 96
NUM_HEADS = 3
K_SP = 576
TOPK = 64
HEAD_DIM = QK_DIM // NUM_HEADS
SC_SCALE = HEAD_DIM ** (-0.5)
HW = H * W
N = K_SP * TOPK
BN = B * N

NC, NS = 2, 16
NW = NC * NS

_HB1 = 8
_T1 = _HB1 * W


def _k1_body(x_ref, lnw_ref, lnb_ref, qw_ref, kw_ref, vw_ref, qk_ref, v_ref):
    x2 = jnp.reshape(x_ref[0], (C, _T1))
    xt = jnp.transpose(x2, (1, 0))
    mu = jnp.mean(xt, axis=1, keepdims=True)
    var = jnp.mean((xt - mu) ** 2, axis=1, keepdims=True)
    xn = (xt - mu) * lax.rsqrt(var + 1e-6)
    xn = xn * lnw_ref[:] + lnb_ref[:]
    dn = (((1,), (1,)), ((), ()))
    q = lax.dot_general(xn, qw_ref[:], dn, preferred_element_type=jnp.float32)
    k = lax.dot_general(xn, kw_ref[:], dn, preferred_element_type=jnp.float32)
    v = lax.dot_general(xn, vw_ref[:], dn, preferred_element_type=jnp.float32)
    z32 = jnp.zeros((_T1, 32), jnp.float32)
    qk_ref[...] = jnp.concatenate([q, z32, k, z32], axis=1)
    v_ref[...] = jnp.concatenate([v, z32], axis=1)


def _ln_qkv(x, ln_w2, ln_b2, q_w, k_w, v_w):
    grid = (B, H // _HB1)
    return pl.pallas_call(
        _k1_body,
        grid=grid,
        in_specs=[
            pl.BlockSpec((1, C, _HB1, W), lambda b, t: (b, 0, t, 0)),
            pl.BlockSpec((1, C), lambda b, t: (0, 0)),
            pl.BlockSpec((1, C), lambda b, t: (0, 0)),
            pl.BlockSpec((C, C), lambda b, t: (0, 0)),
            pl.BlockSpec((C, C), lambda b, t: (0, 0)),
            pl.BlockSpec((C, C), lambda b, t: (0, 0)),
        ],
        out_specs=[
            pl.BlockSpec((_T1, 256), lambda b, t: (b * (H // _HB1) + t, 0)),
            pl.BlockSpec((_T1, 128), lambda b, t: (b * (H // _HB1) + t, 0)),
        ],
        out_shape=[
            jax.ShapeDtypeStruct((B * HW, 256), jnp.float32),
            jax.ShapeDtypeStruct((B * HW, 128), jnp.float32),
        ],
        compiler_params=pltpu.CompilerParams(
            dimension_semantics=("parallel", "parallel")),
    )(x, ln_w2, ln_b2, q_w, k_w, v_w)


_G_ROWS = BN // NW
_G_CH = 96


def _k2_body(qk_hbm, v_hbm, gidx_hbm, qkg_hbm, vg_hbm, idx_v, qbuf, vbuf, sem):
    wid = lax.axis_index("s") * NC + lax.axis_index("c")
    base = wid * _G_ROWS
    pltpu.sync_copy(gidx_hbm.at[pl.ds(base, _G_ROWS)], idx_v)

    def chunk(i, carry):
        off = i * _G_CH
        pltpu.async_copy(qk_hbm.at[idx_v.at[pl.ds(off, _G_CH)]], qbuf, sem).wait()
        pltpu.sync_copy(qbuf, qkg_hbm.at[pl.ds(base + off, _G_CH)])
        pltpu.async_copy(v_hbm.at[idx_v.at[pl.ds(off, _G_CH)]], vbuf, sem).wait()
        pltpu.sync_copy(vbuf, vg_hbm.at[pl.ds(base + off, _G_CH)])
        return carry

    lax.fori_loop(0, _G_ROWS // _G_CH, chunk, 0)


def _sc_gather(qk_tab, v_tab, gidx):
    mesh = plsc.VectorSubcoreMesh(core_axis_name="c", subcore_axis_name="s")
    f = pl.kernel(
        _k2_body,
        out_type=[
            jax.ShapeDtypeStruct((BN, 256), jnp.float32),
            jax.ShapeDtypeStruct((BN, 128), jnp.float32),
        ],
        mesh=mesh,
        scratch_types=[
            pltpu.VMEM((_G_ROWS,), jnp.int32),
            pltpu.VMEM((_G_CH, 256), jnp.float32),
            pltpu.VMEM((_G_CH, 128), jnp.float32),
            pltpu.SemaphoreType.DMA,
        ],
        compiler_params=pltpu.CompilerParams(use_tc_tiling_on_sc=True),
    )
    return f(qk_tab, v_tab, gidx)


_GSP = 4
_RWS = _GSP * TOPK


def _k3_body(qk_ref, v_ref, sims_ref, out_ref, pen_sc):
    X = qk_ref[...]
    XV = v_ref[...]
    sT = jnp.transpose(sims_ref[0], (1, 0))
    ones1 = jnp.ones((_RWS, 1), jnp.float32)

    @pl.when(pl.program_id(0) == 0)
    def _():
        rb = lax.shift_right_logical(
            lax.broadcasted_iota(jnp.int32, (_RWS, _RWS), 0), 6)
        cb = lax.shift_right_logical(
            lax.broadcasted_iota(jnp.int32, (_RWS, _RWS), 1), 6)
        pen_sc[...] = jnp.where(rb == cb, 0.0, -1e30)

    pen = pen_sc[...]
    outs = []
    dn = (((1,), (1,)), ((), ()))
    for h in range(NUM_HEADS):
        q = X[:, h * HEAD_DIM:(h + 1) * HEAD_DIM]
        k = X[:, 128 + h * HEAD_DIM:128 + (h + 1) * HEAD_DIM]
        v = XV[:, h * HEAD_DIM:(h + 1) * HEAD_DIM]
        qn = jnp.sum(q * q, axis=1, keepdims=True)
        kn = jnp.sum(k * k, axis=1, keepdims=True)
        qa = jnp.concatenate([-2.0 * q, ones1], axis=1)
        ka = jnp.concatenate([k, kn], axis=1)
        d2k = lax.dot_general(qa, ka, dn,
                              preferred_element_type=jnp.float32)
        d2 = d2k + qn
        dist = jnp.sqrt(jnp.maximum(d2, 1e-12))
        e = jnp.exp(pen - SC_SCALE * dist)
        vwa = jnp.concatenate([v * sT, ones1],
                              axis=1).astype(jnp.bfloat16)
        dn2 = (((1,), (0,)), ((), ()))
        os = lax.dot_general(e.astype(jnp.bfloat16), vwa, dn2,
                             preferred_element_type=jnp.float32)
        outs.append(os[:, 0:HEAD_DIM] * (sT / os[:, HEAD_DIM:HEAD_DIM + 1]))
    z31 = jnp.zeros((_RWS, 31), jnp.float32)
    out_ref[...] = jnp.concatenate(outs + [ones1, z31], axis=1)


def _attention(qk_g, v_g, sims2):
    grid = (BN // _RWS,)
    return pl.pallas_call(
        _k3_body,
        grid=grid,
        in_specs=[
            pl.BlockSpec((_RWS, 256), lambda i: (i, 0)),
            pl.BlockSpec((_RWS, 128), lambda i: (i, 0)),
            pl.BlockSpec((1, 1, _RWS), lambda i: (i, 0, 0)),
        ],
        out_specs=pl.BlockSpec((_RWS, 128), lambda i: (i, 0)),
        out_shape=jax.ShapeDtypeStruct((BN, 128), jnp.float32),
        scratch_shapes=[pltpu.VMEM((_RWS, _RWS), jnp.float32)],
        compiler_params=pltpu.CompilerParams(
            dimension_semantics=("arbitrary",)),
    )(qk_g, v_g, sims2)


_R = 12288
_NRANGE = HW // _R
_S_TOK = N // NS
_S_CH = 128
_ZROWS = 32


def _k4_body(tok_hbm, gidx_hbm, acc_hbm,
             idxa_v, cidx, cdst, cidx_c, cdst_c, bufa, zbuf,
             sema, semz, acc_sp):
    c = lax.axis_index("c")
    s = lax.axis_index("s")
    base_tok = c * N + s * _S_TOK

    with jax.named_scope("k4_init"):
        def zfill(i, carry):
            def zf2(j, carry2):
                zbuf[i, pl.ds(j * 16, 16)] = jnp.zeros((16,), jnp.float32)
                return carry2
            lax.fori_loop(0, 128 // 16, zf2, 0)
            return carry
        lax.fori_loop(0, _ZROWS, zfill, 0)

    pltpu.sync_copy(gidx_hbm.at[pl.ds(base_tok, _S_TOK)], idxa_v)
    iota16 = lax.iota(jnp.int32, 16)

    def one_range(rr, carry):
        r0g = c * HW + rr * _R
        with jax.named_scope("k4_zero"):
            def za(i, carry2):
                pltpu.async_copy(
                    zbuf,
                    acc_sp.at[pl.ds(s * (_R // NS) + i * _ZROWS, _ZROWS)],
                    semz)
                return carry2
            lax.fori_loop(0, _R // NS // _ZROWS, za, 0)

        with jax.named_scope("k4_compact"):
            def cp(i, off):
                vv = idxa_v[pl.ds(i * 16, 16)] - r0g
                msk = (vv >= 0) & (vv < _R)
                rows = base_tok + i * 16 + iota16
                plsc.store_compressed(cidx.at[pl.ds(off, 16)], rows, mask=msk)
                plsc.store_compressed(cdst.at[pl.ds(off, 16)], vv, mask=msk)
                pc = plsc.all_reduce_population_count(msk)
                return off + pc[0]
            nc = lax.fori_loop(0, _S_TOK // 16, cp, 0)
            def pad(k, carry2):
                cidx[pl.ds(nc + k * 16, 16)] = jnp.full((16,), base_tok,
                                                        jnp.int32)
                cdst[pl.ds(nc + k * 16, 16)] = jnp.full((16,), _R, jnp.int32)
                return carry2
            lax.fori_loop(0, _S_CH // 16, pad, 0)

        with jax.named_scope("k4_zdrain"):
            def zd(i, carry2):
                pltpu.make_async_copy(
                    zbuf, acc_sp.at[pl.ds(s * (_R // NS), _ZROWS)],
                    semz).wait()
                return carry2
            lax.fori_loop(0, _R // NS // _ZROWS, zd, 0)
        plsc.subcore_barrier()

        with jax.named_scope("k4_chunks"):
            nch = lax.shift_right_logical(nc + (_S_CH - 1), 7)

            def chunk(j, carry2):
                def mv(k, carry3):
                    cidx_c[pl.ds(k * 16, 16)] = (
                        cidx[pl.ds(j * _S_CH + k * 16, 16)])
                    cdst_c[pl.ds(k * 16, 16)] = (
                        cdst[pl.ds(j * _S_CH + k * 16, 16)])
                    return carry3
                lax.fori_loop(0, _S_CH // 16, mv, 0)
                pltpu.async_copy(tok_hbm.at[cidx_c], bufa, sema).wait()
                pltpu.sync_copy(bufa, acc_sp.at[cdst_c], add=True)
                return carry2
            lax.fori_loop(0, nch, chunk, 0)
        plsc.subcore_barrier()

        with jax.named_scope("k4_out"):
            pltpu.sync_copy(acc_sp.at[pl.ds(s * (_R // NS), _R // NS)],
                            acc_hbm.at[pl.ds(r0g + s * (_R // NS), _R // NS)])
        plsc.subcore_barrier()
        return carry

    lax.fori_loop(0, _NRANGE, one_range, 0)


def _sc_scatter(out_tok, gidx):
    mesh = plsc.VectorSubcoreMesh(core_axis_name="c", subcore_axis_name="s")
    f = pl.kernel(
        _k4_body,
        out_type=jax.ShapeDtypeStruct((B * HW, 128), jnp.float32),
        mesh=mesh,
        scratch_types=[
            pltpu.VMEM((_S_TOK,), jnp.int32),
            pltpu.VMEM((_S_TOK + _S_CH,), jnp.int32),
            pltpu.VMEM((_S_TOK + _S_CH,), jnp.int32),
            pltpu.VMEM((_S_CH,), jnp.int32),
            pltpu.VMEM((_S_CH,), jnp.int32),
            pltpu.VMEM((_S_CH, 128), jnp.float32),
            pltpu.VMEM((_ZROWS, 128), jnp.float32),
            pltpu.SemaphoreType.DMA,
            pltpu.SemaphoreType.DMA,
            pltpu.VMEM_SHARED((_R + 16, 128), jnp.float32),
        ],
        compiler_params=pltpu.CompilerParams(use_tc_tiling_on_sc=True,
                                             needs_layout_passes=False),
    )
    return f(out_tok, gidx)


_T5 = 12288
_HB5 = _T5 // W


def _k5_body(acc_ref, v_ref, out_ref):
    a = acc_ref[...]
    ct = a[:, C:C + 1]
    v = v_ref[:, 0:C]
    mean = a[:, 0:C] / jnp.maximum(ct, 1.0)
    res = jnp.where(ct > 1e-5, mean, v)
    rT = jnp.transpose(res, (1, 0))
    for hb in range(_HB5):
        out_ref[0, :, hb, :] = rT[:, hb * W:(hb + 1) * W]


def _merge(acc, v_tok):
    grid = (B * HW // _T5,)
    nh = H // _HB5
    return pl.pallas_call(
        _k5_body,
        grid=grid,
        in_specs=[
            pl.BlockSpec((_T5, 128), lambda t: (t, 0)),
            pl.BlockSpec((_T5, 128), lambda t: (t, 0)),
        ],
        out_specs=pl.BlockSpec((1, C, _HB5, W), lambda t: (t // nh, 0, t % nh, 0)),
        out_shape=jax.ShapeDtypeStruct((B, C, H, W), jnp.float32),
        compiler_params=pltpu.CompilerParams(
            dimension_semantics=("arbitrary",)),
    )(acc, v_tok)


@jax.jit
def _run(x, sims, ln_w, ln_b, q_w, k_w, v_w, indices):
    qk_tok, v_tok = _ln_qkv(x, ln_w.reshape(1, C), ln_b.reshape(1, C),
                            q_w, k_w, v_w)
    gidx = (indices.reshape(B, N)
            + (jnp.arange(B, dtype=jnp.int32) * HW)[:, None]).reshape(BN)
    qk_g, v_g = _sc_gather(qk_tok, v_tok, gidx)
    out_tok = _attention(qk_g, v_g, sims.reshape(BN // _RWS, 1, _RWS))
    acc = _sc_scatter(out_tok, gidx)
    return _merge(acc, v_tok)


def kernel(x, sims, mask, ln_w, ln_b, q_w, k_w, v_w, indices, labels,
           num_spixels):
    del mask, labels, num_spixels
    return _run(x, sims, ln_w, ln_b, q_w, k_w, v_w, indices)
